# baseline (speedup 1.0000x reference)
import math

import jax
import jax.numpy as jnp
import numpy as np
from jax.experimental import pallas as pl
from jax.experimental.pallas import tpu as pltpu

_LMAX = 6
_NB = 20
_RCUT = 10.0
_C = 4
_B = 16
_N = 16384
_NLM = (_LMAX + 1) ** 2
_F = _NB * _NLM
_S = _B * _C
_T = 1024
_G = _N // _T

_HIGH = jax.lax.Precision.DEFAULT


def _build_consts():
    R = np.zeros((_F, _NLM), np.float32)
    Tm = np.zeros((_F, _NB), np.float32)
    for f in range(_F):
        R[f, f // _NB] = 1.0
        Tm[f, f % _NB] = 1.0
    idx = []
    for l in range(_LMAX + 1):
        for c in range(_C):
            for n in range(_NB):
                for m in range(-l, l + 1):
                    idx.append(c * _F + (l * l + l + m) * _NB + n)
    return R, Tm, np.array(idx, np.int32)


_Rmat, _Tmat, _OUT_IDX = _build_consts()


def _sph_kernel(ct_ref, nb_ref, el_ref, out_ref):
    g = pl.program_id(0)
    xyz = jnp.swapaxes(ct_ref[...], 0, 1)
    x = xyz[0:1, :]
    y = xyz[1:2, :]
    z = xyz[2:3, :]
    r = jnp.sqrt(x * x + y * y + z * z)
    inv_r = 1.0 / jnp.maximum(r, 1e-12)
    xn = x * inv_r
    yn = y * inv_r
    zn = z * inv_r
    ctheta = zn
    st = jnp.sqrt(jnp.maximum(1.0 - zn * zn, 1e-12))
    inv_rho = 1.0 / jnp.maximum(jnp.sqrt(xn * xn + yn * yn), 1e-20)
    c1 = xn * inv_rho
    s1 = yn * inv_rho
    cm = [None, c1]
    sm = [None, s1]
    for m in range(2, _LMAX + 1):
        cm.append(cm[m - 1] * c1 - sm[m - 1] * s1)
        sm.append(sm[m - 1] * c1 + cm[m - 1] * s1)
    P = {(0, 0): jnp.ones_like(z)}
    for m in range(1, _LMAX + 1):
        P[(m, m)] = (-(2 * m - 1.0)) * st * P[(m - 1, m - 1)]
    for m in range(0, _LMAX):
        P[(m + 1, m)] = (2 * m + 1.0) * ctheta * P[(m, m)]
    for m in range(0, _LMAX + 1):
        for l in range(m + 2, _LMAX + 1):
            P[(l, m)] = ((2 * l - 1.0) * ctheta * P[(l - 1, m)]
                         - (l + m - 1.0) * P[(l - 2, m)]) * (1.0 / (l - m))
    rows = []
    for l in range(_LMAX + 1):
        for m in range(-l, l + 1):
            am = abs(m)
            nlm = math.sqrt((2 * l + 1) / (4.0 * math.pi)
                            * math.factorial(l - am) / math.factorial(l + am))
            nlm *= math.sqrt(4.0 * math.pi)
            if m < 0:
                rows.append((math.sqrt(2.0) * nlm) * P[(l, am)] * sm[am])
            elif m == 0:
                rows.append(nlm * P[(l, 0)])
            else:
                rows.append((math.sqrt(2.0) * nlm) * P[(l, am)] * cm[am])
    sh_T = jnp.concatenate(rows, axis=0)
    r_c = jnp.maximum(r, 1e-6)
    n_col = (jax.lax.broadcasted_iota(jnp.int32, (_NB, 1), 0)
             .astype(jnp.float32) + 1.0)
    rad_T = (math.sqrt(2.0 / _RCUT) * jnp.sin(n_col * (math.pi / _RCUT) * r_c)
             / r_c)
    A = jnp.broadcast_to(sh_T[:, None, :], (_NLM, _NB, _T)).reshape(_F, _T)
    Bv = pltpu.repeat(rad_T, _NLM, axis=0)
    pw = A * Bv
    seg_c = nb_ref[0] * _C + el_ref[0]
    hot = (seg_c == jax.lax.broadcasted_iota(jnp.int32, (1, _S), 1)
           ).astype(jnp.float32)
    contrib = jax.lax.dot(pw, hot, precision=_HIGH,
                          preferred_element_type=jnp.float32)

    @pl.when(g == 0)
    def _():
        out_ref[...] = contrib

    @pl.when(g > 0)
    def _():
        out_ref[...] = out_ref[...] + contrib


def _pallas_core(ct, nbr, elr, interpret=False):
    return pl.pallas_call(
        _sph_kernel,
        grid=(_G,),
        in_specs=[
            pl.BlockSpec((_T, 3), lambda g: (g, 0)),
            pl.BlockSpec((1, _T, 1), lambda g: (g, 0, 0)),
            pl.BlockSpec((1, _T, 1), lambda g: (g, 0, 0)),
        ],
        out_specs=pl.BlockSpec((_F, _S), lambda g: (0, 0)),
        out_shape=jax.ShapeDtypeStruct((_F, _S), jnp.float32),
        compiler_params=pltpu.CompilerParams(
            dimension_semantics=("arbitrary",)),
        interpret=interpret,
    )(ct, nbr, elr)


def kernel(x_coords_B_N3, x_elements_B_N, nb_indices):
    nbr = nb_indices.astype(jnp.int32).reshape(_G, _T, 1)
    elr = x_elements_B_N.astype(jnp.int32).reshape(_G, _T, 1)
    out_raw = _pallas_core(x_coords_B_N3, nbr, elr)
    per = out_raw.reshape(_NLM, _NB, _B, _C).transpose(2, 3, 0, 1)
    blocks = []
    for l in range(_LMAX + 1):
        w = 2 * l + 1
        blk = per[:, :, l * l:l * l + w, :]
        blocks.append(blk.transpose(0, 1, 3, 2).reshape(_B, _C * _NB * w))
    return jnp.concatenate(blocks, axis=-1)


# speedup vs baseline: 1.1081x; 1.1081x over previous
import math

import jax
import jax.numpy as jnp
import numpy as np
from jax.experimental import pallas as pl
from jax.experimental.pallas import tpu as pltpu

_LMAX = 6
_NB = 20
_RCUT = 10.0
_C = 4
_B = 16
_N = 16384
_NLM = (_LMAX + 1) ** 2
_F = _NB * _NLM
_S = _B * _C
_T = 1024
_G = _N // _T

_HIGH = jax.lax.Precision.DEFAULT


def _build_consts():
    R = np.zeros((_F, _NLM), np.float32)
    Tm = np.zeros((_F, _NB), np.float32)
    for f in range(_F):
        R[f, f // _NB] = 1.0
        Tm[f, f % _NB] = 1.0
    idx = []
    for l in range(_LMAX + 1):
        for c in range(_C):
            for n in range(_NB):
                for m in range(-l, l + 1):
                    idx.append(c * _F + (l * l + l + m) * _NB + n)
    return R, Tm, np.array(idx, np.int32)


_Rmat, _Tmat, _OUT_IDX = _build_consts()


def _sph_kernel(ct_ref, nb_ref, el_ref, out_ref):
    g = pl.program_id(0)
    xyz = ct_ref[0]
    x = xyz[0:1, :]
    y = xyz[1:2, :]
    z = xyz[2:3, :]
    r = jnp.sqrt(x * x + y * y + z * z)
    inv_r = 1.0 / jnp.maximum(r, 1e-12)
    xn = x * inv_r
    yn = y * inv_r
    zn = z * inv_r
    ctheta = zn
    st = jnp.sqrt(jnp.maximum(1.0 - zn * zn, 1e-12))
    inv_rho = 1.0 / jnp.maximum(jnp.sqrt(xn * xn + yn * yn), 1e-20)
    c1 = xn * inv_rho
    s1 = yn * inv_rho
    cm = [None, c1]
    sm = [None, s1]
    for m in range(2, _LMAX + 1):
        cm.append(cm[m - 1] * c1 - sm[m - 1] * s1)
        sm.append(sm[m - 1] * c1 + cm[m - 1] * s1)
    P = {(0, 0): jnp.ones_like(z)}
    for m in range(1, _LMAX + 1):
        P[(m, m)] = (-(2 * m - 1.0)) * st * P[(m - 1, m - 1)]
    for m in range(0, _LMAX):
        P[(m + 1, m)] = (2 * m + 1.0) * ctheta * P[(m, m)]
    for m in range(0, _LMAX + 1):
        for l in range(m + 2, _LMAX + 1):
            P[(l, m)] = ((2 * l - 1.0) * ctheta * P[(l - 1, m)]
                         - (l + m - 1.0) * P[(l - 2, m)]) * (1.0 / (l - m))
    rows = []
    for l in range(_LMAX + 1):
        for m in range(-l, l + 1):
            am = abs(m)
            nlm = math.sqrt((2 * l + 1) / (4.0 * math.pi)
                            * math.factorial(l - am) / math.factorial(l + am))
            nlm *= math.sqrt(4.0 * math.pi)
            if m < 0:
                rows.append((math.sqrt(2.0) * nlm) * P[(l, am)] * sm[am])
            elif m == 0:
                rows.append(nlm * P[(l, 0)])
            else:
                rows.append((math.sqrt(2.0) * nlm) * P[(l, am)] * cm[am])
    sh_T = jnp.concatenate(rows, axis=0)
    r_c = jnp.maximum(r, 1e-6)
    inv_rc = (math.sqrt(2.0 / _RCUT)) / r_c
    n_col = (jax.lax.broadcasted_iota(jnp.int32, (_NB, 1), 0)
             .astype(jnp.float32) + 1.0)
    rad_T = jnp.sin(n_col * ((math.pi / _RCUT) * r_c)) * inv_rc
    A = jnp.broadcast_to(sh_T[:, None, :], (_NLM, _NB, _T)).reshape(_F, _T)
    Bv = jnp.broadcast_to(rad_T[None, :, :], (_NLM, _NB, _T)).reshape(_F, _T)
    pw = A * Bv
    seg_c = nb_ref[0] * _C + el_ref[0]
    hot = (seg_c == jax.lax.broadcasted_iota(jnp.int32, (1, _S), 1)
           ).astype(jnp.float32)
    contrib = jax.lax.dot(pw, hot, precision=_HIGH,
                          preferred_element_type=jnp.float32)

    @pl.when(g == 0)
    def _():
        out_ref[...] = contrib

    @pl.when(g > 0)
    def _():
        out_ref[...] = out_ref[...] + contrib


def _pallas_core(ct, nbr, elr, interpret=False):
    return pl.pallas_call(
        _sph_kernel,
        grid=(_G,),
        in_specs=[
            pl.BlockSpec((1, 3, _T), lambda g: (g, 0, 0)),
            pl.BlockSpec((1, _T, 1), lambda g: (g, 0, 0)),
            pl.BlockSpec((1, _T, 1), lambda g: (g, 0, 0)),
        ],
        out_specs=pl.BlockSpec((_F, _S), lambda g: (0, 0)),
        out_shape=jax.ShapeDtypeStruct((_F, _S), jnp.float32),
        compiler_params=pltpu.CompilerParams(
            dimension_semantics=("arbitrary",)),
        interpret=interpret,
    )(ct, nbr, elr)


def kernel(x_coords_B_N3, x_elements_B_N, nb_indices):
    ct = x_coords_B_N3.T.reshape(3, _G, _T).transpose(1, 0, 2)
    nbr = nb_indices.astype(jnp.int32).reshape(_G, _T, 1)
    elr = x_elements_B_N.astype(jnp.int32).reshape(_G, _T, 1)
    out_raw = _pallas_core(ct, nbr, elr)
    per = out_raw.reshape(_NLM, _NB, _B, _C).transpose(2, 3, 0, 1)
    blocks = []
    for l in range(_LMAX + 1):
        w = 2 * l + 1
        blk = per[:, :, l * l:l * l + w, :]
        blocks.append(blk.transpose(0, 1, 3, 2).reshape(_B, _C * _NB * w))
    return jnp.concatenate(blocks, axis=-1)


# speedup vs baseline: 1.3518x; 1.2199x over previous
import math

import jax
import jax.numpy as jnp
import numpy as np
from jax.experimental import pallas as pl
from jax.experimental.pallas import tpu as pltpu

_LMAX = 6
_NB = 20
_RCUT = 10.0
_C = 4
_B = 16
_N = 16384
_NLM = (_LMAX + 1) ** 2
_F = _NB * _NLM
_S = _B * _C
_T = 1024
_G = _N // _T

_HIGH = jax.lax.Precision.DEFAULT


def _build_consts():
    R = np.zeros((_F, _NLM), np.float32)
    Tm = np.zeros((_F, _NB), np.float32)
    for f in range(_F):
        R[f, f // _NB] = 1.0
        Tm[f, f % _NB] = 1.0
    idx = []
    for l in range(_LMAX + 1):
        for c in range(_C):
            for n in range(_NB):
                for m in range(-l, l + 1):
                    idx.append(c * _F + (l * l + l + m) * _NB + n)
    return R, Tm, np.array(idx, np.int32)


_Rmat, _Tmat, _OUT_IDX = _build_consts()


def _sph_kernel(ct_ref, segc_ref, out_ref):
    g = pl.program_id(0)
    xyz = ct_ref[0]
    x = xyz[0:1, :]
    y = xyz[1:2, :]
    z = xyz[2:3, :]
    r = jnp.sqrt(x * x + y * y + z * z)
    inv_r = 1.0 / jnp.maximum(r, 1e-12)
    xn = x * inv_r
    yn = y * inv_r
    zn = z * inv_r
    ctheta = zn
    st = jnp.sqrt(jnp.maximum(1.0 - zn * zn, 1e-12))
    inv_rho = 1.0 / jnp.maximum(jnp.sqrt(xn * xn + yn * yn), 1e-20)
    c1 = xn * inv_rho
    s1 = yn * inv_rho
    cm = [None, c1]
    sm = [None, s1]
    for m in range(2, _LMAX + 1):
        cm.append(cm[m - 1] * c1 - sm[m - 1] * s1)
        sm.append(sm[m - 1] * c1 + cm[m - 1] * s1)
    P = {(0, 0): jnp.ones_like(z)}
    for m in range(1, _LMAX + 1):
        P[(m, m)] = (-(2 * m - 1.0)) * st * P[(m - 1, m - 1)]
    for m in range(0, _LMAX):
        P[(m + 1, m)] = (2 * m + 1.0) * ctheta * P[(m, m)]
    for m in range(0, _LMAX + 1):
        for l in range(m + 2, _LMAX + 1):
            P[(l, m)] = ((2 * l - 1.0) * ctheta * P[(l - 1, m)]
                         - (l + m - 1.0) * P[(l - 2, m)]) * (1.0 / (l - m))
    rows = []
    for l in range(_LMAX + 1):
        for m in range(-l, l + 1):
            am = abs(m)
            nlm = math.sqrt((2 * l + 1) / (4.0 * math.pi)
                            * math.factorial(l - am) / math.factorial(l + am))
            nlm *= math.sqrt(4.0 * math.pi)
            if m < 0:
                rows.append((math.sqrt(2.0) * nlm) * P[(l, am)] * sm[am])
            elif m == 0:
                rows.append(nlm * P[(l, 0)])
            else:
                rows.append((math.sqrt(2.0) * nlm) * P[(l, am)] * cm[am])
    sh_T = jnp.concatenate(rows, axis=0)
    r_c = jnp.maximum(r, 1e-6)
    inv_rc = (math.sqrt(2.0 / _RCUT)) / r_c
    n_col = (jax.lax.broadcasted_iota(jnp.int32, (_NB, 1), 0)
             .astype(jnp.float32) + 1.0)
    rad_T = jnp.sin(n_col * ((math.pi / _RCUT) * r_c)) * inv_rc
    A = jnp.broadcast_to(sh_T[:, None, :], (_NLM, _NB, _T)).reshape(_F, _T)
    Bv = jnp.broadcast_to(rad_T[None, :, :], (_NLM, _NB, _T)).reshape(_F, _T)
    pw = A * Bv
    seg_c = segc_ref[0]
    hot = (seg_c == jax.lax.broadcasted_iota(jnp.int32, (1, _S), 1)
           ).astype(jnp.float32)
    contrib = jax.lax.dot(pw, hot, precision=_HIGH,
                          preferred_element_type=jnp.float32)

    @pl.when(g == 0)
    def _():
        out_ref[...] = contrib

    @pl.when(g > 0)
    def _():
        out_ref[...] = out_ref[...] + contrib


def _pallas_core(ct, segc, interpret=False):
    return pl.pallas_call(
        _sph_kernel,
        grid=(_G,),
        in_specs=[
            pl.BlockSpec((1, 3, _T), lambda g: (g, 0, 0)),
            pl.BlockSpec((1, _T, 1), lambda g: (g, 0, 0)),
        ],
        out_specs=pl.BlockSpec((_F, _S), lambda g: (0, 0)),
        out_shape=jax.ShapeDtypeStruct((_F, _S), jnp.float32),
        compiler_params=pltpu.CompilerParams(
            dimension_semantics=("arbitrary",)),
        interpret=interpret,
    )(ct, segc)


def kernel(x_coords_B_N3, x_elements_B_N, nb_indices):
    seg = nb_indices.astype(jnp.int32) * _C + x_elements_B_N.astype(jnp.int32)
    ct = x_coords_B_N3.T.reshape(3, _G, _T).transpose(1, 0, 2)
    segc = seg.reshape(_G, _T, 1)
    out_raw = _pallas_core(ct, segc)
    per = out_raw.reshape(_NLM, _NB, _B, _C).transpose(2, 3, 0, 1)
    blocks = []
    for l in range(_LMAX + 1):
        w = 2 * l + 1
        blk = per[:, :, l * l:l * l + w, :]
        blocks.append(blk.transpose(0, 1, 3, 2).reshape(_B, _C * _NB * w))
    return jnp.concatenate(blocks, axis=-1)


# speedup vs baseline: 1.4412x; 1.0661x over previous
import math

import jax
import jax.numpy as jnp
import numpy as np
from jax.experimental import pallas as pl
from jax.experimental.pallas import tpu as pltpu

_LMAX = 6
_NB = 20
_RCUT = 10.0
_C = 4
_B = 16
_N = 16384
_NLM = (_LMAX + 1) ** 2
_F = _NB * _NLM
_S = _B * _C
_T = 2048
_G = _N // _T

_HIGH = jax.lax.Precision.DEFAULT


def _build_consts():
    R = np.zeros((_F, _NLM), np.float32)
    Tm = np.zeros((_F, _NB), np.float32)
    for f in range(_F):
        R[f, f // _NB] = 1.0
        Tm[f, f % _NB] = 1.0
    idx = []
    for l in range(_LMAX + 1):
        for c in range(_C):
            for n in range(_NB):
                for m in range(-l, l + 1):
                    idx.append(c * _F + (l * l + l + m) * _NB + n)
    return R, Tm, np.array(idx, np.int32)


_Rmat, _Tmat, _OUT_IDX = _build_consts()


def _sph_kernel(ct_ref, segc_ref, out_ref):
    g = pl.program_id(0)
    xyz = ct_ref[0]
    x = xyz[0:1, :]
    y = xyz[1:2, :]
    z = xyz[2:3, :]
    r = jnp.sqrt(x * x + y * y + z * z)
    inv_r = 1.0 / jnp.maximum(r, 1e-12)
    xn = x * inv_r
    yn = y * inv_r
    zn = z * inv_r
    ctheta = zn
    st = jnp.sqrt(jnp.maximum(1.0 - zn * zn, 1e-12))
    inv_rho = 1.0 / jnp.maximum(jnp.sqrt(xn * xn + yn * yn), 1e-20)
    c1 = xn * inv_rho
    s1 = yn * inv_rho
    cm = [None, c1]
    sm = [None, s1]
    for m in range(2, _LMAX + 1):
        cm.append(cm[m - 1] * c1 - sm[m - 1] * s1)
        sm.append(sm[m - 1] * c1 + cm[m - 1] * s1)
    P = {(0, 0): jnp.ones_like(z)}
    for m in range(1, _LMAX + 1):
        P[(m, m)] = (-(2 * m - 1.0)) * st * P[(m - 1, m - 1)]
    for m in range(0, _LMAX):
        P[(m + 1, m)] = (2 * m + 1.0) * ctheta * P[(m, m)]
    for m in range(0, _LMAX + 1):
        for l in range(m + 2, _LMAX + 1):
            P[(l, m)] = ((2 * l - 1.0) * ctheta * P[(l - 1, m)]
                         - (l + m - 1.0) * P[(l - 2, m)]) * (1.0 / (l - m))
    rows = []
    for l in range(_LMAX + 1):
        for m in range(-l, l + 1):
            am = abs(m)
            nlm = math.sqrt((2 * l + 1) / (4.0 * math.pi)
                            * math.factorial(l - am) / math.factorial(l + am))
            nlm *= math.sqrt(4.0 * math.pi)
            if m < 0:
                rows.append((math.sqrt(2.0) * nlm) * P[(l, am)] * sm[am])
            elif m == 0:
                rows.append(nlm * P[(l, 0)])
            else:
                rows.append((math.sqrt(2.0) * nlm) * P[(l, am)] * cm[am])
    sh_T = jnp.concatenate(rows, axis=0)
    r_c = jnp.maximum(r, 1e-6)
    inv_rc = (math.sqrt(2.0 / _RCUT)) / r_c
    n_col = (jax.lax.broadcasted_iota(jnp.int32, (_NB, 1), 0)
             .astype(jnp.float32) + 1.0)
    rad_T = jnp.sin(n_col * ((math.pi / _RCUT) * r_c)) * inv_rc
    A = jnp.broadcast_to(sh_T[:, None, :], (_NLM, _NB, _T)).reshape(_F, _T)
    Bv = jnp.broadcast_to(rad_T[None, :, :], (_NLM, _NB, _T)).reshape(_F, _T)
    pw = A * Bv
    seg_c = segc_ref[0]
    hot = (seg_c == jax.lax.broadcasted_iota(jnp.int32, (1, _S), 1)
           ).astype(jnp.float32)
    contrib = jax.lax.dot(pw, hot, precision=_HIGH,
                          preferred_element_type=jnp.float32)

    @pl.when(g == 0)
    def _():
        out_ref[...] = contrib

    @pl.when(g > 0)
    def _():
        out_ref[...] = out_ref[...] + contrib


def _pallas_core(ct, segc, interpret=False):
    return pl.pallas_call(
        _sph_kernel,
        grid=(_G,),
        in_specs=[
            pl.BlockSpec((1, 3, _T), lambda g: (g, 0, 0)),
            pl.BlockSpec((1, _T, 1), lambda g: (g, 0, 0)),
        ],
        out_specs=pl.BlockSpec((_F, _S), lambda g: (0, 0)),
        out_shape=jax.ShapeDtypeStruct((_F, _S), jnp.float32),
        compiler_params=pltpu.CompilerParams(
            dimension_semantics=("arbitrary",)),
        interpret=interpret,
    )(ct, segc)


def kernel(x_coords_B_N3, x_elements_B_N, nb_indices):
    seg = nb_indices.astype(jnp.int32) * _C + x_elements_B_N.astype(jnp.int32)
    ct = x_coords_B_N3.T.reshape(3, _G, _T).transpose(1, 0, 2)
    segc = seg.reshape(_G, _T, 1)
    out_raw = _pallas_core(ct, segc)
    per = out_raw.reshape(_NLM, _NB, _B, _C).transpose(2, 3, 0, 1)
    blocks = []
    for l in range(_LMAX + 1):
        w = 2 * l + 1
        blk = per[:, :, l * l:l * l + w, :]
        blocks.append(blk.transpose(0, 1, 3, 2).reshape(_B, _C * _NB * w))
    return jnp.concatenate(blocks, axis=-1)
